# Initial kernel scaffold; baseline (speedup 1.0000x reference)
import jax
import jax.numpy as jnp
from jax.experimental import pallas as pl


def kernel(x, edge_index, W1, b1, W2, b2, Wh, bh):
    raise NotImplementedError("write your pallas kernel here")


# speedup vs baseline: 15.3431x; 15.3431x over previous
import functools

import jax
import jax.numpy as jnp
from jax import lax
from jax.experimental import pallas as pl
from jax.experimental.pallas import tpu as pltpu
from jax.experimental.pallas import tpu_sc as plsc

N = 10000
NP = 10240
E = 320000
F_IN = 128
HID = 64
A_OUT = 8

NC = 2
NS = 16
NW = NC * NS
CH = 128
NCHUNK = -(-E // (NW * CH))
E_PAD = NW * CH * NCHUNK
PER_TILE = CH * NCHUNK
RPT = NP // NS

_mesh = plsc.VectorSubcoreMesh(core_axis_name="c", subcore_axis_name="s")
_sc_params = pltpu.CompilerParams(use_tc_tiling_on_sc=False)


@functools.partial(
    pl.kernel,
    mesh=_mesh,
    out_type=jax.ShapeDtypeStruct((NC, NP), jnp.float32),
    compiler_params=_sc_params,
    scratch_types=[
        pltpu.VMEM((CH,), jnp.int32),
        pltpu.VMEM((CH,), jnp.float32),
        pltpu.VMEM_SHARED((NP,), jnp.float32),
    ],
)
def _deg_kernel(dst_hbm, zeros_hbm, out_hbm, dst_v, ones_v, acc):
    cid = lax.axis_index("c")
    sid = lax.axis_index("s")
    wid = sid * NC + cid

    for i in range(CH // 16):
        ones_v[pl.ds(i * 16, 16)] = jnp.full((16,), 1.0, jnp.float32)
    pltpu.sync_copy(zeros_hbm, acc.at[pl.ds(sid * RPT, RPT)])
    plsc.subcore_barrier()

    def body(c, carry):
        base = wid * PER_TILE + c * CH
        pltpu.sync_copy(dst_hbm.at[pl.ds(base, CH)], dst_v)
        pltpu.sync_copy(ones_v, acc.at[dst_v], add=True)
        return carry

    lax.fori_loop(0, NCHUNK, body, 0)
    plsc.subcore_barrier()
    pltpu.sync_copy(acc.at[pl.ds(sid * RPT, RPT)],
                    out_hbm.at[cid, pl.ds(sid * RPT, RPT)])


@functools.partial(
    pl.kernel,
    mesh=_mesh,
    out_type=jax.ShapeDtypeStruct((NC, NP, HID), jnp.float32),
    compiler_params=_sc_params,
    scratch_types=[
        pltpu.VMEM((CH,), jnp.int32),
        pltpu.VMEM((CH,), jnp.int32),
        pltpu.VMEM((CH, HID), jnp.float32),
        pltpu.VMEM_SHARED((NP, HID), jnp.float32),
        pltpu.SemaphoreType.DMA,
    ],
)
def _agg_kernel(src_hbm, dst_hbm, g_hbm, zeros_hbm, out_hbm,
                src_v, dst_v, rows_v, acc, sem):
    cid = lax.axis_index("c")
    sid = lax.axis_index("s")
    wid = sid * NC + cid

    pltpu.sync_copy(zeros_hbm, acc.at[pl.ds(sid * RPT, RPT)])
    plsc.subcore_barrier()

    def body(c, carry):
        base = wid * PER_TILE + c * CH
        pltpu.sync_copy(src_hbm.at[pl.ds(base, CH)], src_v)
        pltpu.sync_copy(dst_hbm.at[pl.ds(base, CH)], dst_v)
        pltpu.async_copy(g_hbm.at[src_v], rows_v, sem).wait()
        pltpu.sync_copy(rows_v, acc.at[dst_v], add=True)
        return carry

    lax.fori_loop(0, NCHUNK, body, 0)
    plsc.subcore_barrier()
    pltpu.sync_copy(acc.at[pl.ds(sid * RPT, RPT)],
                    out_hbm.at[cid, pl.ds(sid * RPT, RPT)])


R = 1000


def _dense1_body(dega, degb, x, w1, g_out, dis_out):
    dis = lax.rsqrt(dega[...] + degb[...] + 1.0)
    h = jnp.dot(x[...], w1[...], preferred_element_type=jnp.float32)
    g_out[...] = h * dis
    dis_out[...] = dis


def _dense1(dega, degb, x, w1):
    return pl.pallas_call(
        _dense1_body,
        grid=(N // R,),
        in_specs=[
            pl.BlockSpec((R, 1), lambda i: (i, 0)),
            pl.BlockSpec((R, 1), lambda i: (i, 0)),
            pl.BlockSpec((R, F_IN), lambda i: (i, 0)),
            pl.BlockSpec((F_IN, HID), lambda i: (0, 0)),
        ],
        out_specs=[
            pl.BlockSpec((R, HID), lambda i: (i, 0)),
            pl.BlockSpec((R, 1), lambda i: (i, 0)),
        ],
        out_shape=[
            jax.ShapeDtypeStruct((N, HID), jnp.float32),
            jax.ShapeDtypeStruct((N, 1), jnp.float32),
        ],
    )(dega, degb, x, w1)


def _dense2_body(agga, aggb, g, dis, b, w, out):
    t = jnp.maximum(dis[...] * (agga[...] + aggb[...] + g[...]) + b[...], 0.0)
    out[...] = jnp.dot(t, w[...], preferred_element_type=jnp.float32) * dis[...]


def _dense2(agga, aggb, g, dis, b, w):
    return pl.pallas_call(
        _dense2_body,
        grid=(N // R,),
        in_specs=[
            pl.BlockSpec((R, HID), lambda i: (i, 0)),
            pl.BlockSpec((R, HID), lambda i: (i, 0)),
            pl.BlockSpec((R, HID), lambda i: (i, 0)),
            pl.BlockSpec((R, 1), lambda i: (i, 0)),
            pl.BlockSpec((1, HID), lambda i: (0, 0)),
            pl.BlockSpec((HID, HID), lambda i: (0, 0)),
        ],
        out_specs=pl.BlockSpec((R, HID), lambda i: (i, 0)),
        out_shape=jax.ShapeDtypeStruct((N, HID), jnp.float32),
    )(agga, aggb, g, dis, b, w)


def _dense3_body(agga, aggb, g, dis, b, w, bh, out):
    t = jnp.maximum(dis[...] * (agga[...] + aggb[...] + g[...]) + b[...], 0.0)
    out[...] = jnp.dot(t, w[...], preferred_element_type=jnp.float32) + bh[...]


def _dense3(agga, aggb, g, dis, b, w, bh):
    return pl.pallas_call(
        _dense3_body,
        grid=(N // R,),
        in_specs=[
            pl.BlockSpec((R, HID), lambda i: (i, 0)),
            pl.BlockSpec((R, HID), lambda i: (i, 0)),
            pl.BlockSpec((R, HID), lambda i: (i, 0)),
            pl.BlockSpec((R, 1), lambda i: (i, 0)),
            pl.BlockSpec((1, HID), lambda i: (0, 0)),
            pl.BlockSpec((HID, A_OUT), lambda i: (0, 0)),
            pl.BlockSpec((1, A_OUT), lambda i: (0, 0)),
        ],
        out_specs=pl.BlockSpec((R, A_OUT), lambda i: (i, 0)),
        out_shape=jax.ShapeDtypeStruct((N, A_OUT), jnp.float32),
    )(agga, aggb, g, dis, b, w, bh)


def kernel(x, edge_index, W1, b1, W2, b2, Wh, bh):
    pad = E_PAD - E
    srcp = jnp.concatenate([edge_index[0], jnp.zeros((pad,), jnp.int32)])
    dstp = jnp.concatenate([edge_index[1], jnp.full((pad,), N, jnp.int32)])
    zeros_row = jnp.zeros((RPT,), jnp.float32)
    zeros_mat = jnp.zeros((RPT, HID), jnp.float32)

    degp = _deg_kernel(dstp, zeros_row)
    dega = degp[0, :N].reshape(N, 1)
    degb = degp[1, :N].reshape(N, 1)
    g1, dis = _dense1(dega, degb, x, W1)

    agg1 = _agg_kernel(srcp, dstp, g1, zeros_mat)
    g2 = _dense2(agg1[0, :N], agg1[1, :N], g1, dis,
                 b1.reshape(1, HID), W2)

    agg2 = _agg_kernel(srcp, dstp, g2, zeros_mat)
    return _dense3(agg2[0, :N], agg2[1, :N], g2, dis,
                   b2.reshape(1, HID), Wh, bh.reshape(1, A_OUT))


# speedup vs baseline: 16.9091x; 1.1021x over previous
import functools

import jax
import jax.numpy as jnp
from jax import lax
from jax.experimental import pallas as pl
from jax.experimental.pallas import tpu as pltpu
from jax.experimental.pallas import tpu_sc as plsc

N = 10000
NP = 10240
E = 320000
F_IN = 128
HID = 64
A_OUT = 8

NC = 2
NS = 16
NW = NC * NS
CH = 128
NBUF = 8
NCHUNK = 80
E_PAD = NW * CH * NCHUNK
PER_TILE = CH * NCHUNK
RPT = NP // NS
NGRP = NCHUNK // NBUF

_mesh = plsc.VectorSubcoreMesh(core_axis_name="c", subcore_axis_name="s")
_sc_params = pltpu.CompilerParams(use_tc_tiling_on_sc=False)


@functools.partial(
    pl.kernel,
    mesh=_mesh,
    out_type=jax.ShapeDtypeStruct((NC, NP), jnp.float32),
    compiler_params=_sc_params,
    scratch_types=[
        pltpu.VMEM((NCHUNK, CH), jnp.int32),
        pltpu.VMEM((CH,), jnp.float32),
        pltpu.VMEM_SHARED((NP,), jnp.float32),
        pltpu.SemaphoreType.DMA,
    ],
)
def _deg_kernel(dst_hbm, zeros_hbm, out_hbm, dst_v, ones_v, acc, sem):
    cid = lax.axis_index("c")
    sid = lax.axis_index("s")
    wid = sid * NC + cid

    for i in range(CH // 16):
        ones_v[pl.ds(i * 16, 16)] = jnp.full((16,), 1.0, jnp.float32)
    pltpu.sync_copy(zeros_hbm, acc.at[pl.ds(sid * RPT, RPT)])
    pltpu.sync_copy(dst_hbm.at[wid], dst_v)
    plsc.subcore_barrier()

    def body(c, carry):
        pltpu.async_copy(ones_v, acc.at[dst_v.at[c]], sem, add=True)

        @pl.when(c >= NBUF)
        def _():
            pltpu.make_async_copy(ones_v, acc.at[dst_v.at[0]], sem).wait()

        return carry

    lax.fori_loop(0, NCHUNK, body, 0)
    for _ in range(NBUF):
        pltpu.make_async_copy(ones_v, acc.at[dst_v.at[0]], sem).wait()
    plsc.subcore_barrier()
    pltpu.sync_copy(acc.at[pl.ds(sid * RPT, RPT)],
                    out_hbm.at[cid, pl.ds(sid * RPT, RPT)])


@functools.partial(
    pl.kernel,
    mesh=_mesh,
    out_type=jax.ShapeDtypeStruct((NC, NP, HID), jnp.float32),
    compiler_params=_sc_params,
    scratch_types=(
        [pltpu.VMEM((NCHUNK, CH), jnp.int32)] * 2
        + [pltpu.VMEM((CH, HID), jnp.float32)] * NBUF
        + [pltpu.VMEM_SHARED((NP, HID), jnp.float32)]
        + [pltpu.SemaphoreType.DMA] * (2 * NBUF)
    ),
)
def _agg_kernel(src_hbm, dst_hbm, g_hbm, zeros_hbm, out_hbm, *refs):
    src_v, dst_v = refs[0], refs[1]
    rows = refs[2:2 + NBUF]
    acc = refs[2 + NBUF]
    gsem = refs[3 + NBUF:3 + 2 * NBUF]
    ssem = refs[3 + 2 * NBUF:3 + 3 * NBUF]
    cid = lax.axis_index("c")
    sid = lax.axis_index("s")
    wid = sid * NC + cid

    pltpu.sync_copy(zeros_hbm, acc.at[pl.ds(sid * RPT, RPT)])
    pltpu.sync_copy(src_hbm.at[wid], src_v)
    pltpu.sync_copy(dst_hbm.at[wid], dst_v)
    plsc.subcore_barrier()

    for b in range(NBUF):
        pltpu.async_copy(g_hbm.at[src_v.at[b]], rows[b], gsem[b])

    def group(gi, carry):
        c0 = gi * NBUF
        for b in range(NBUF):
            pltpu.make_async_copy(g_hbm.at[src_v.at[c0 + b]], rows[b],
                                  gsem[b]).wait()
            pltpu.async_copy(rows[b], acc.at[dst_v.at[c0 + b]], ssem[b],
                             add=True)
        for b in range(NBUF):
            pltpu.make_async_copy(rows[b], acc.at[dst_v.at[c0 + b]],
                                  ssem[b]).wait()
            pltpu.async_copy(g_hbm.at[src_v.at[c0 + NBUF + b]], rows[b],
                             gsem[b])
        return carry

    lax.fori_loop(0, NGRP - 1, group, 0)

    c0 = NCHUNK - NBUF
    for b in range(NBUF):
        pltpu.make_async_copy(g_hbm.at[src_v.at[c0 + b]], rows[b],
                              gsem[b]).wait()
        pltpu.async_copy(rows[b], acc.at[dst_v.at[c0 + b]], ssem[b], add=True)
    for b in range(NBUF):
        pltpu.make_async_copy(rows[b], acc.at[dst_v.at[c0 + b]],
                              ssem[b]).wait()

    plsc.subcore_barrier()
    pltpu.sync_copy(acc.at[pl.ds(sid * RPT, RPT)],
                    out_hbm.at[cid, pl.ds(sid * RPT, RPT)])


R = 1000


def _dense1_body(dega, degb, x, w1, g_out, dis_out):
    dis = lax.rsqrt(dega[...] + degb[...] + 1.0)
    h = jnp.dot(x[...], w1[...], preferred_element_type=jnp.float32)
    g_out[...] = h * dis
    dis_out[...] = dis


def _dense1(dega, degb, x, w1):
    return pl.pallas_call(
        _dense1_body,
        grid=(N // R,),
        in_specs=[
            pl.BlockSpec((R, 1), lambda i: (i, 0)),
            pl.BlockSpec((R, 1), lambda i: (i, 0)),
            pl.BlockSpec((R, F_IN), lambda i: (i, 0)),
            pl.BlockSpec((F_IN, HID), lambda i: (0, 0)),
        ],
        out_specs=[
            pl.BlockSpec((R, HID), lambda i: (i, 0)),
            pl.BlockSpec((R, 1), lambda i: (i, 0)),
        ],
        out_shape=[
            jax.ShapeDtypeStruct((N, HID), jnp.float32),
            jax.ShapeDtypeStruct((N, 1), jnp.float32),
        ],
    )(dega, degb, x, w1)


def _dense2_body(agga, aggb, g, dis, b, w, out):
    t = jnp.maximum(dis[...] * (agga[...] + aggb[...] + g[...]) + b[...], 0.0)
    out[...] = jnp.dot(t, w[...], preferred_element_type=jnp.float32) * dis[...]


def _dense2(agga, aggb, g, dis, b, w):
    return pl.pallas_call(
        _dense2_body,
        grid=(N // R,),
        in_specs=[
            pl.BlockSpec((R, HID), lambda i: (i, 0)),
            pl.BlockSpec((R, HID), lambda i: (i, 0)),
            pl.BlockSpec((R, HID), lambda i: (i, 0)),
            pl.BlockSpec((R, 1), lambda i: (i, 0)),
            pl.BlockSpec((1, HID), lambda i: (0, 0)),
            pl.BlockSpec((HID, HID), lambda i: (0, 0)),
        ],
        out_specs=pl.BlockSpec((R, HID), lambda i: (i, 0)),
        out_shape=jax.ShapeDtypeStruct((N, HID), jnp.float32),
    )(agga, aggb, g, dis, b, w)


def _dense3_body(agga, aggb, g, dis, b, w, bh, out):
    t = jnp.maximum(dis[...] * (agga[...] + aggb[...] + g[...]) + b[...], 0.0)
    out[...] = jnp.dot(t, w[...], preferred_element_type=jnp.float32) + bh[...]


def _dense3(agga, aggb, g, dis, b, w, bh):
    return pl.pallas_call(
        _dense3_body,
        grid=(N // R,),
        in_specs=[
            pl.BlockSpec((R, HID), lambda i: (i, 0)),
            pl.BlockSpec((R, HID), lambda i: (i, 0)),
            pl.BlockSpec((R, HID), lambda i: (i, 0)),
            pl.BlockSpec((R, 1), lambda i: (i, 0)),
            pl.BlockSpec((1, HID), lambda i: (0, 0)),
            pl.BlockSpec((HID, A_OUT), lambda i: (0, 0)),
            pl.BlockSpec((1, A_OUT), lambda i: (0, 0)),
        ],
        out_specs=pl.BlockSpec((R, A_OUT), lambda i: (i, 0)),
        out_shape=jax.ShapeDtypeStruct((N, A_OUT), jnp.float32),
    )(agga, aggb, g, dis, b, w, bh)


def kernel(x, edge_index, W1, b1, W2, b2, Wh, bh):
    pad = E_PAD - E
    srcp = jnp.concatenate([edge_index[0], jnp.zeros((pad,), jnp.int32)])
    dstp = jnp.concatenate([edge_index[1], jnp.full((pad,), N, jnp.int32)])
    src3 = srcp.reshape(NW, NCHUNK, CH)
    dst3 = dstp.reshape(NW, NCHUNK, CH)
    zeros_row = jnp.zeros((RPT,), jnp.float32)
    zeros_mat = jnp.zeros((RPT, HID), jnp.float32)

    degp = _deg_kernel(dst3, zeros_row)
    dega = degp[0, :N].reshape(N, 1)
    degb = degp[1, :N].reshape(N, 1)
    g1, dis = _dense1(dega, degb, x, W1)

    agg1 = _agg_kernel(src3, dst3, g1, zeros_mat)
    g2 = _dense2(agg1[0, :N], agg1[1, :N], g1, dis,
                 b1.reshape(1, HID), W2)

    agg2 = _agg_kernel(src3, dst3, g2, zeros_mat)
    return _dense3(agg2[0, :N], agg2[1, :N], g2, dis,
                   b2.reshape(1, HID), Wh, bh.reshape(1, A_OUT))


# speedup vs baseline: 42.0120x; 2.4846x over previous
import functools

import jax
import jax.numpy as jnp
from jax import lax
from jax.experimental import pallas as pl
from jax.experimental.pallas import tpu as pltpu
from jax.experimental.pallas import tpu_sc as plsc

N = 10000
NP = 10240
E = 320000
F_IN = 128
HID = 64
A_OUT = 8

NC = 2
NS = 16
NW = NC * NS
CH = 128
NBUF = 8
NCHUNK = 80
E_PAD = NW * CH * NCHUNK
PER_TILE = CH * NCHUNK
RPT = NP // NS
NGRP = NCHUNK // NBUF

_mesh = plsc.VectorSubcoreMesh(core_axis_name="c", subcore_axis_name="s")
_sc_params = pltpu.CompilerParams(use_tc_tiling_on_sc=False)


@functools.partial(
    pl.kernel,
    mesh=_mesh,
    out_type=jax.ShapeDtypeStruct((NC, NP), jnp.float32),
    compiler_params=_sc_params,
    scratch_types=[
        pltpu.VMEM((NCHUNK, CH), jnp.int32),
        pltpu.VMEM((CH,), jnp.float32),
        pltpu.VMEM_SHARED((NP,), jnp.float32),
        pltpu.SemaphoreType.DMA,
    ],
)
def _deg_kernel(dst_hbm, zeros_hbm, out_hbm, dst_v, ones_v, acc, sem):
    cid = lax.axis_index("c")
    sid = lax.axis_index("s")
    wid = sid * NC + cid

    for i in range(CH // 16):
        ones_v[pl.ds(i * 16, 16)] = jnp.full((16,), 1.0, jnp.float32)
    pltpu.sync_copy(zeros_hbm, acc.at[pl.ds(sid * RPT, RPT)])
    pltpu.sync_copy(dst_hbm.at[wid], dst_v)
    plsc.subcore_barrier()

    def body(c, carry):
        pltpu.async_copy(ones_v, acc.at[dst_v.at[c]], sem, add=True)

        @pl.when(c >= NBUF)
        def _():
            pltpu.make_async_copy(ones_v, acc.at[dst_v.at[0]], sem).wait()

        return carry

    lax.fori_loop(0, NCHUNK, body, 0)
    for _ in range(NBUF):
        pltpu.make_async_copy(ones_v, acc.at[dst_v.at[0]], sem).wait()
    plsc.subcore_barrier()
    pltpu.sync_copy(acc.at[pl.ds(sid * RPT, RPT)],
                    out_hbm.at[cid, pl.ds(sid * RPT, RPT)])


@functools.partial(
    pl.kernel,
    mesh=_mesh,
    out_type=jax.ShapeDtypeStruct((NC, NP, HID), jnp.bfloat16),
    compiler_params=_sc_params,
    scratch_types=(
        [pltpu.VMEM((NCHUNK, CH), jnp.int32)] * 2
        + [pltpu.VMEM((CH, HID), jnp.bfloat16)] * NBUF
        + [pltpu.VMEM_SHARED((N, HID), jnp.bfloat16)]
        + [pltpu.VMEM_SHARED((NP, HID), jnp.bfloat16)]
        + [pltpu.SemaphoreType.DMA] * (2 * NBUF)
    ),
)
def _agg_kernel(src_hbm, dst_hbm, g_hbm, zeros_hbm, out_hbm, *refs):
    src_v, dst_v = refs[0], refs[1]
    rows = refs[2:2 + NBUF]
    gsh = refs[2 + NBUF]
    acc = refs[3 + NBUF]
    gsem = refs[4 + NBUF:4 + 2 * NBUF]
    ssem = refs[4 + 2 * NBUF:4 + 3 * NBUF]
    cid = lax.axis_index("c")
    sid = lax.axis_index("s")
    wid = sid * NC + cid

    pltpu.sync_copy(zeros_hbm, acc.at[pl.ds(sid * RPT, RPT)])
    pltpu.sync_copy(g_hbm.at[pl.ds(sid * (N // NS), N // NS)],
                    gsh.at[pl.ds(sid * (N // NS), N // NS)])
    pltpu.sync_copy(src_hbm.at[wid], src_v)
    pltpu.sync_copy(dst_hbm.at[wid], dst_v)
    plsc.subcore_barrier()

    for b in range(NBUF):
        pltpu.async_copy(gsh.at[src_v.at[b]], rows[b], gsem[b])

    def group(gi, carry):
        c0 = gi * NBUF
        for b in range(NBUF):
            pltpu.make_async_copy(gsh.at[src_v.at[c0 + b]], rows[b],
                                  gsem[b]).wait()
            pltpu.async_copy(rows[b], acc.at[dst_v.at[c0 + b]], ssem[b],
                             add=True)
        for b in range(NBUF):
            pltpu.make_async_copy(rows[b], acc.at[dst_v.at[c0 + b]],
                                  ssem[b]).wait()
            pltpu.async_copy(gsh.at[src_v.at[c0 + NBUF + b]], rows[b],
                             gsem[b])
        return carry

    lax.fori_loop(0, NGRP - 1, group, 0)

    c0 = NCHUNK - NBUF
    for b in range(NBUF):
        pltpu.make_async_copy(gsh.at[src_v.at[c0 + b]], rows[b],
                              gsem[b]).wait()
        pltpu.async_copy(rows[b], acc.at[dst_v.at[c0 + b]], ssem[b], add=True)
    for b in range(NBUF):
        pltpu.make_async_copy(rows[b], acc.at[dst_v.at[c0 + b]],
                              ssem[b]).wait()

    plsc.subcore_barrier()
    pltpu.sync_copy(acc.at[pl.ds(sid * RPT, RPT)],
                    out_hbm.at[cid, pl.ds(sid * RPT, RPT)])


R = 1000


def _dense1_body(dega, degb, x, w1, g_out, gb_out, dis_out):
    dis = lax.rsqrt(dega[...] + degb[...] + 1.0)
    h = jnp.dot(x[...], w1[...], preferred_element_type=jnp.float32)
    g = h * dis
    g_out[...] = g
    gb_out[...] = g.astype(jnp.bfloat16)
    dis_out[...] = dis


def _dense1(dega, degb, x, w1):
    return pl.pallas_call(
        _dense1_body,
        grid=(N // R,),
        in_specs=[
            pl.BlockSpec((R, 1), lambda i: (i, 0)),
            pl.BlockSpec((R, 1), lambda i: (i, 0)),
            pl.BlockSpec((R, F_IN), lambda i: (i, 0)),
            pl.BlockSpec((F_IN, HID), lambda i: (0, 0)),
        ],
        out_specs=[
            pl.BlockSpec((R, HID), lambda i: (i, 0)),
            pl.BlockSpec((R, HID), lambda i: (i, 0)),
            pl.BlockSpec((R, 1), lambda i: (i, 0)),
        ],
        out_shape=[
            jax.ShapeDtypeStruct((N, HID), jnp.float32),
            jax.ShapeDtypeStruct((N, HID), jnp.bfloat16),
            jax.ShapeDtypeStruct((N, 1), jnp.float32),
        ],
    )(dega, degb, x, w1)


def _dense2_body(agga, aggb, g, dis, b, w, out, gb_out):
    agg = agga[...].astype(jnp.float32) + aggb[...].astype(jnp.float32)
    t = jnp.maximum(dis[...] * (agg + g[...]) + b[...], 0.0)
    g2 = jnp.dot(t, w[...], preferred_element_type=jnp.float32) * dis[...]
    out[...] = g2
    gb_out[...] = g2.astype(jnp.bfloat16)


def _dense2(agga, aggb, g, dis, b, w):
    return pl.pallas_call(
        _dense2_body,
        grid=(N // R,),
        in_specs=[
            pl.BlockSpec((R, HID), lambda i: (i, 0)),
            pl.BlockSpec((R, HID), lambda i: (i, 0)),
            pl.BlockSpec((R, HID), lambda i: (i, 0)),
            pl.BlockSpec((R, 1), lambda i: (i, 0)),
            pl.BlockSpec((1, HID), lambda i: (0, 0)),
            pl.BlockSpec((HID, HID), lambda i: (0, 0)),
        ],
        out_specs=[
            pl.BlockSpec((R, HID), lambda i: (i, 0)),
            pl.BlockSpec((R, HID), lambda i: (i, 0)),
        ],
        out_shape=[
            jax.ShapeDtypeStruct((N, HID), jnp.float32),
            jax.ShapeDtypeStruct((N, HID), jnp.bfloat16),
        ],
    )(agga, aggb, g, dis, b, w)


def _dense3_body(agga, aggb, g, dis, b, w, bh, out):
    agg = agga[...].astype(jnp.float32) + aggb[...].astype(jnp.float32)
    t = jnp.maximum(dis[...] * (agg + g[...]) + b[...], 0.0)
    out[...] = jnp.dot(t, w[...], preferred_element_type=jnp.float32) + bh[...]


def _dense3(agga, aggb, g, dis, b, w, bh):
    return pl.pallas_call(
        _dense3_body,
        grid=(N // R,),
        in_specs=[
            pl.BlockSpec((R, HID), lambda i: (i, 0)),
            pl.BlockSpec((R, HID), lambda i: (i, 0)),
            pl.BlockSpec((R, HID), lambda i: (i, 0)),
            pl.BlockSpec((R, 1), lambda i: (i, 0)),
            pl.BlockSpec((1, HID), lambda i: (0, 0)),
            pl.BlockSpec((HID, A_OUT), lambda i: (0, 0)),
            pl.BlockSpec((1, A_OUT), lambda i: (0, 0)),
        ],
        out_specs=pl.BlockSpec((R, A_OUT), lambda i: (i, 0)),
        out_shape=jax.ShapeDtypeStruct((N, A_OUT), jnp.float32),
    )(agga, aggb, g, dis, b, w, bh)


def kernel(x, edge_index, W1, b1, W2, b2, Wh, bh):
    pad = E_PAD - E
    srcp = jnp.concatenate([edge_index[0], jnp.zeros((pad,), jnp.int32)])
    dstp = jnp.concatenate([edge_index[1], jnp.full((pad,), N, jnp.int32)])
    src3 = srcp.reshape(NW, NCHUNK, CH)
    dst3 = dstp.reshape(NW, NCHUNK, CH)
    zeros_row = jnp.zeros((RPT,), jnp.float32)
    zeros_mat = jnp.zeros((RPT, HID), jnp.bfloat16)

    degp = _deg_kernel(dst3, zeros_row)
    dega = degp[0, :N].reshape(N, 1)
    degb = degp[1, :N].reshape(N, 1)
    g1, g1b, dis = _dense1(dega, degb, x, W1)

    agg1 = _agg_kernel(src3, dst3, g1b, zeros_mat)
    g2, g2b = _dense2(agg1[0, :N], agg1[1, :N], g1, dis,
                      b1.reshape(1, HID), W2)

    agg2 = _agg_kernel(src3, dst3, g2b, zeros_mat)
    return _dense3(agg2[0, :N], agg2[1, :N], g2, dis,
                   b2.reshape(1, HID), Wh, bh.reshape(1, A_OUT))


# speedup vs baseline: 42.4215x; 1.0097x over previous
import functools

import jax
import jax.numpy as jnp
from jax import lax
from jax.experimental import pallas as pl
from jax.experimental.pallas import tpu as pltpu
from jax.experimental.pallas import tpu_sc as plsc

N = 10000
NP = 10240
E = 320000
F_IN = 128
HID = 64
A_OUT = 8

NC = 2
NS = 16
NW = NC * NS
CH = 125
NBUF = 8
NCHUNK = 80
PER_TILE = CH * NCHUNK
RPT = NP // NS
NGRP = NCHUNK // NBUF

_mesh = plsc.VectorSubcoreMesh(core_axis_name="c", subcore_axis_name="s")
_sc_params = pltpu.CompilerParams(use_tc_tiling_on_sc=False)


@functools.partial(
    pl.kernel,
    mesh=_mesh,
    out_type=jax.ShapeDtypeStruct((NC, NP), jnp.float32),
    compiler_params=_sc_params,
    scratch_types=[
        pltpu.VMEM((NCHUNK, CH), jnp.int32),
        pltpu.VMEM((128,), jnp.float32),
        pltpu.VMEM_SHARED((NP,), jnp.float32),
        pltpu.SemaphoreType.DMA,
    ],
)
def _deg_kernel(dst_hbm, zeros_hbm, out_hbm, dst_v, ones_v, acc, sem):
    cid = lax.axis_index("c")
    sid = lax.axis_index("s")
    wid = sid * NC + cid

    for i in range(128 // 16):
        ones_v[pl.ds(i * 16, 16)] = jnp.full((16,), 1.0, jnp.float32)
    pltpu.sync_copy(zeros_hbm, acc.at[pl.ds(sid * RPT, RPT)])
    pltpu.sync_copy(dst_hbm.at[wid], dst_v)
    plsc.subcore_barrier()

    def body(c, carry):
        pltpu.async_copy(ones_v.at[pl.ds(0, CH)], acc.at[dst_v.at[c]], sem,
                         add=True)

        @pl.when(c >= NBUF)
        def _():
            pltpu.make_async_copy(ones_v.at[pl.ds(0, CH)], acc.at[dst_v.at[0]],
                                  sem).wait()

        return carry

    lax.fori_loop(0, NCHUNK, body, 0)
    for _ in range(NBUF):
        pltpu.make_async_copy(ones_v.at[pl.ds(0, CH)], acc.at[dst_v.at[0]],
                              sem).wait()
    plsc.subcore_barrier()
    pltpu.sync_copy(acc.at[pl.ds(sid * RPT, RPT)],
                    out_hbm.at[cid, pl.ds(sid * RPT, RPT)])


@functools.partial(
    pl.kernel,
    mesh=_mesh,
    out_type=jax.ShapeDtypeStruct((NC, NP, HID), jnp.bfloat16),
    compiler_params=_sc_params,
    scratch_types=(
        [pltpu.VMEM((NCHUNK, CH), jnp.int32)] * 2
        + [pltpu.VMEM((CH, HID), jnp.bfloat16)] * NBUF
        + [pltpu.VMEM_SHARED((N, HID), jnp.bfloat16)]
        + [pltpu.VMEM_SHARED((NP, HID), jnp.bfloat16)]
        + [pltpu.SemaphoreType.DMA] * (2 * NBUF)
    ),
)
def _agg_kernel(src_hbm, dst_hbm, g_hbm, zeros_hbm, out_hbm, *refs):
    src_v, dst_v = refs[0], refs[1]
    rows = refs[2:2 + NBUF]
    gsh = refs[2 + NBUF]
    acc = refs[3 + NBUF]
    gsem = refs[4 + NBUF:4 + 2 * NBUF]
    ssem = refs[4 + 2 * NBUF:4 + 3 * NBUF]
    cid = lax.axis_index("c")
    sid = lax.axis_index("s")
    wid = sid * NC + cid

    pltpu.sync_copy(zeros_hbm, acc.at[pl.ds(sid * RPT, RPT)])
    pltpu.sync_copy(g_hbm.at[pl.ds(sid * (N // NS), N // NS)],
                    gsh.at[pl.ds(sid * (N // NS), N // NS)])
    pltpu.sync_copy(src_hbm.at[wid], src_v)
    pltpu.sync_copy(dst_hbm.at[wid], dst_v)
    plsc.subcore_barrier()

    for b in range(NBUF):
        pltpu.async_copy(gsh.at[src_v.at[b]], rows[b], gsem[b])

    def group(gi, carry):
        c0 = gi * NBUF
        for b in range(NBUF):
            pltpu.make_async_copy(gsh.at[src_v.at[c0 + b]], rows[b],
                                  gsem[b]).wait()
            pltpu.async_copy(rows[b], acc.at[dst_v.at[c0 + b]], ssem[b],
                             add=True)
        for b in range(NBUF):
            pltpu.make_async_copy(rows[b], acc.at[dst_v.at[c0 + b]],
                                  ssem[b]).wait()
            pltpu.async_copy(gsh.at[src_v.at[c0 + NBUF + b]], rows[b],
                             gsem[b])
        return carry

    lax.fori_loop(0, NGRP - 1, group, 0)

    c0 = NCHUNK - NBUF
    for b in range(NBUF):
        pltpu.make_async_copy(gsh.at[src_v.at[c0 + b]], rows[b],
                              gsem[b]).wait()
        pltpu.async_copy(rows[b], acc.at[dst_v.at[c0 + b]], ssem[b], add=True)
    for b in range(NBUF):
        pltpu.make_async_copy(rows[b], acc.at[dst_v.at[c0 + b]],
                              ssem[b]).wait()

    plsc.subcore_barrier()
    pltpu.sync_copy(acc.at[pl.ds(sid * RPT, RPT)],
                    out_hbm.at[cid, pl.ds(sid * RPT, RPT)])


R = 1000


def _mm1_body(x, w1, h_out):
    h_out[...] = jnp.dot(x[...], w1[...], preferred_element_type=jnp.float32)


def _mm1(x, w1):
    return pl.pallas_call(
        _mm1_body,
        grid=(N // R,),
        in_specs=[
            pl.BlockSpec((R, F_IN), lambda i: (i, 0)),
            pl.BlockSpec((F_IN, HID), lambda i: (0, 0)),
        ],
        out_specs=pl.BlockSpec((R, HID), lambda i: (i, 0)),
        out_shape=jax.ShapeDtypeStruct((N, HID), jnp.float32),
    )(x, w1)


def _scale1_body(dega, degb, h, g_out, gb_out, dis_out):
    dis = lax.rsqrt(dega[...] + degb[...] + 1.0)
    g = h[...] * dis
    g_out[...] = g
    gb_out[...] = g.astype(jnp.bfloat16)
    dis_out[...] = dis


def _scale1(dega, degb, h):
    return pl.pallas_call(
        _scale1_body,
        grid=(N // R,),
        in_specs=[
            pl.BlockSpec((R, 1), lambda i: (i, 0)),
            pl.BlockSpec((R, 1), lambda i: (i, 0)),
            pl.BlockSpec((R, HID), lambda i: (i, 0)),
        ],
        out_specs=[
            pl.BlockSpec((R, HID), lambda i: (i, 0)),
            pl.BlockSpec((R, HID), lambda i: (i, 0)),
            pl.BlockSpec((R, 1), lambda i: (i, 0)),
        ],
        out_shape=[
            jax.ShapeDtypeStruct((N, HID), jnp.float32),
            jax.ShapeDtypeStruct((N, HID), jnp.bfloat16),
            jax.ShapeDtypeStruct((N, 1), jnp.float32),
        ],
    )(dega, degb, h)


def _dense2_body(agga, aggb, g, dis, b, w, out, gb_out):
    agg = agga[...].astype(jnp.float32) + aggb[...].astype(jnp.float32)
    t = jnp.maximum(dis[...] * (agg + g[...]) + b[...], 0.0)
    g2 = jnp.dot(t, w[...], preferred_element_type=jnp.float32) * dis[...]
    out[...] = g2
    gb_out[...] = g2.astype(jnp.bfloat16)


def _dense2(agga, aggb, g, dis, b, w):
    return pl.pallas_call(
        _dense2_body,
        grid=(N // R,),
        in_specs=[
            pl.BlockSpec((R, HID), lambda i: (i, 0)),
            pl.BlockSpec((R, HID), lambda i: (i, 0)),
            pl.BlockSpec((R, HID), lambda i: (i, 0)),
            pl.BlockSpec((R, 1), lambda i: (i, 0)),
            pl.BlockSpec((1, HID), lambda i: (0, 0)),
            pl.BlockSpec((HID, HID), lambda i: (0, 0)),
        ],
        out_specs=[
            pl.BlockSpec((R, HID), lambda i: (i, 0)),
            pl.BlockSpec((R, HID), lambda i: (i, 0)),
        ],
        out_shape=[
            jax.ShapeDtypeStruct((N, HID), jnp.float32),
            jax.ShapeDtypeStruct((N, HID), jnp.bfloat16),
        ],
    )(agga, aggb, g, dis, b, w)


def _dense3_body(agga, aggb, g, dis, b, w, bh, out):
    agg = agga[...].astype(jnp.float32) + aggb[...].astype(jnp.float32)
    t = jnp.maximum(dis[...] * (agg + g[...]) + b[...], 0.0)
    out[...] = jnp.dot(t, w[...], preferred_element_type=jnp.float32) + bh[...]


def _dense3(agga, aggb, g, dis, b, w, bh):
    return pl.pallas_call(
        _dense3_body,
        grid=(N // R,),
        in_specs=[
            pl.BlockSpec((R, HID), lambda i: (i, 0)),
            pl.BlockSpec((R, HID), lambda i: (i, 0)),
            pl.BlockSpec((R, HID), lambda i: (i, 0)),
            pl.BlockSpec((R, 1), lambda i: (i, 0)),
            pl.BlockSpec((1, HID), lambda i: (0, 0)),
            pl.BlockSpec((HID, A_OUT), lambda i: (0, 0)),
            pl.BlockSpec((1, A_OUT), lambda i: (0, 0)),
        ],
        out_specs=pl.BlockSpec((R, A_OUT), lambda i: (i, 0)),
        out_shape=jax.ShapeDtypeStruct((N, A_OUT), jnp.float32),
    )(agga, aggb, g, dis, b, w, bh)


def kernel(x, edge_index, W1, b1, W2, b2, Wh, bh):
    src3 = edge_index[0].reshape(NW, NCHUNK, CH)
    dst3 = edge_index[1].reshape(NW, NCHUNK, CH)
    zeros_row = jnp.zeros((RPT,), jnp.float32)
    zeros_mat = jnp.zeros((RPT, HID), jnp.bfloat16)

    degp = _deg_kernel(dst3, zeros_row)
    h1 = _mm1(x, W1)
    dega = degp[0, :N].reshape(N, 1)
    degb = degp[1, :N].reshape(N, 1)
    g1, g1b, dis = _scale1(dega, degb, h1)

    agg1 = _agg_kernel(src3, dst3, g1b, zeros_mat)
    g2, g2b = _dense2(agg1[0, :N], agg1[1, :N], g1, dis,
                      b1.reshape(1, HID), W2)

    agg2 = _agg_kernel(src3, dst3, g2b, zeros_mat)
    return _dense3(agg2[0, :N], agg2[1, :N], g2, dis,
                   b2.reshape(1, HID), Wh, bh.reshape(1, A_OUT))
